# baseline (speedup 1.0000x reference)
import functools
import jax
import jax.numpy as jnp
from jax import lax
from jax.experimental import pallas as pl
from jax.experimental.pallas import tpu as pltpu
from jax.experimental.pallas import tpu_sc as plsc

N = 10000
E = 320000
CH = 64
NS = 16
NC = 2
NCHUNK = 320
EPT = NCHUNK * CH
E_PAD = EPT * NS
NB_T = NCHUNK // 4
NT8 = NCHUNK // 8
ACC_ROWS = 10240
RPT = ACC_ROWS // NS
TAB_ROWS = 10240
TRPT = TAB_ROWS // NS
TSTAGE = 64
NBLK = 2000
EBLK = 2048


def _leaky(v):
    return jnp.where(v >= 0.0, v, 0.01 * v)


def _sigmoid(v):
    return 1.0 / (1.0 + jnp.exp(-v))


def _pre_node_body(x_ref, h0_ref, c0_ref, wi_t, bi, wh_t, bh, wo_t, bo,
                   ln_g, ln_b, wg_t, bg, uf_t, wx_ref, tab_ref):
    xb = x_ref[...]
    f = _leaky(jnp.dot(xb, wi_t[...], preferred_element_type=jnp.float32) + bi[...])
    f = _leaky(jnp.dot(f, wh_t[...], preferred_element_type=jnp.float32) + bh[...])
    f = jnp.dot(f, wo_t[...], preferred_element_type=jnp.float32) + bo[...]
    mu = jnp.mean(f, axis=-1, keepdims=True)
    var = jnp.mean((f - mu) * (f - mu), axis=-1, keepdims=True)
    xe = (f - mu) / jnp.sqrt(var + 1e-5) * ln_g[...] + ln_b[...]
    wx_ref[...] = jnp.dot(xe, wg_t[...], preferred_element_type=jnp.float32) + bg[...]
    h0b = h0_ref[...]
    hu = jnp.dot(h0b, uf_t[...], preferred_element_type=jnp.float32)
    sc = _sigmoid(c0_ref[...])
    tab_ref[0] = jnp.concatenate([h0b[:, 0:32], hu[:, 0:32], sc[:, 0:32]], axis=-1)
    tab_ref[1] = jnp.concatenate([h0b[:, 32:64], hu[:, 32:64], sc[:, 32:64]], axis=-1)


def _pre_edge_body(ef_ref, w80, b80, w81, b81, efw_ref):
    e8 = ef_ref[...]
    y0 = jnp.dot(e8, w80[...], preferred_element_type=jnp.float32) + b80[...]
    y1 = jnp.dot(e8, w81[...], preferred_element_type=jnp.float32) + b81[...]
    efw_ref[0, 0] = y0[:, 0:128]
    efw_ref[0, 1] = y0[:, 128:256]
    efw_ref[1, 0] = y1[:, 0:128]
    efw_ref[1, 1] = y1[:, 128:256]


def _post_body(acc_ref, wx_ref, ui_t, uo_t, uu_t, w1_t, b1, w2_t, b2, w3_t, b3,
               y_ref):
    a0 = acc_ref[0]
    a1 = acc_ref[1]
    s = jnp.concatenate([a0[:, 0:32], a1[:, 0:32]], axis=-1)
    fc = jnp.concatenate([a0[:, 32:64], a1[:, 32:64]], axis=-1)
    wx = wx_ref[...]
    i_g = _sigmoid(wx[:, 0:64] + jnp.dot(s, ui_t[...], preferred_element_type=jnp.float32))
    o_g = _sigmoid(wx[:, 64:128] + jnp.dot(s, uo_t[...], preferred_element_type=jnp.float32))
    u = jnp.tanh(wx[:, 128:192] + jnp.dot(s, uu_t[...], preferred_element_type=jnp.float32))
    c = fc + i_g * u
    h = o_g * jnp.tanh(c)
    f = _leaky(jnp.dot(h, w1_t[...], preferred_element_type=jnp.float32) + b1[...])
    f = _leaky(jnp.dot(f, w2_t[...], preferred_element_type=jnp.float32) + b2[...])
    y_ref[...] = jnp.dot(f, w3_t[...], preferred_element_type=jnp.float32) + b3[...]


def _sc_body(tab_hbm, src_hbm, dst_hbm, efw_hbm, out_hbm,
             sbuf0, sbuf1, dbuf0, dbuf1, rows0, rows1, efw0, efw1, out0, out1,
             tab_sh, acc_sh, is0, is1, gs0, gs1, es0, es1, ss0, ss1):
    c = lax.axis_index("c")
    s = lax.axis_index("s")
    sbufs = [sbuf0, sbuf1]
    dbufs = [dbuf0, dbuf1]
    rows = [rows0, rows1]
    efws = [efw0, efw1]
    outs = [out0, out1]
    isem = [is0, is1]
    gsem = [gs0, gs1]
    esem = [es0, es1]
    ssem = [ss0, ss1]
    tile_blk = s * NB_T
    tile_edge = s * EPT

    def zrow(i, carry):
        for k in range(4):
            out0[i, pl.ds(16 * k, 16)] = jnp.zeros((16,), jnp.float32)
        return carry

    lax.fori_loop(0, CH, zrow, 0)

    def zcopy(i, carry):
        pltpu.sync_copy(out0, acc_sh.at[pl.ds(s * RPT + i * TSTAGE, TSTAGE)])
        return carry

    lax.fori_loop(0, RPT // TSTAGE, zcopy, 0)

    def stage(i, carry):
        pltpu.sync_copy(tab_hbm.at[c, pl.ds(s * TRPT + i * TSTAGE, TSTAGE)], rows0)
        pltpu.sync_copy(rows0, tab_sh.at[pl.ds(s * TRPT + i * TSTAGE, TSTAGE)])
        return carry

    lax.fori_loop(0, TRPT // TSTAGE, stage, 0)
    plsc.subcore_barrier()

    def idx_descs(gblk, p):
        return (pltpu.make_async_copy(src_hbm.at[gblk], sbufs[p], isem[p]),
                pltpu.make_async_copy(dst_hbm.at[gblk], dbufs[p], isem[p]))

    def gather_desc(p, b, slot):
        return pltpu.make_async_copy(
            tab_sh.at[sbufs[p].at[pl.ds(b * CH, CH)]], rows[slot], gsem[slot])

    def efw_descs(g, slot):
        base8 = tile_edge // 8 + g * (CH // 8)
        return [pltpu.make_async_copy(
            efw_hbm.at[c, p, pl.ds(base8, CH // 8)], efws[slot].at[p],
            esem[slot]) for p in range(2)]

    def scat_desc(p, b, slot):
        return pltpu.make_async_copy(outs[slot], acc_sh.at[dbufs[p].at[b]],
                                     ssem[slot])

    def compute(slot):
        @plsc.parallel_loop(0, CH // 8, unroll=2)
        def erow(i):
            for p2 in range(2):
                for e in range(4):
                    j = i * 8 + p2 * 4 + e
                    for k in range(2):
                        off = 16 * k
                        h0v = rows[slot][j, pl.ds(off, 16)]
                        huv = rows[slot][j, pl.ds(32 + off, 16)]
                        scv = rows[slot][j, pl.ds(64 + off, 16)]
                        ew = efws[slot][p2, i, pl.ds(e * 32 + off, 16)]
                        sg = 1.0 / (1.0 + jnp.exp(-(ew + huv)))
                        outs[slot][j, pl.ds(off, 16)] = h0v
                        outs[slot][j, pl.ds(32 + off, 16)] = sg * scv

    pltpu.sync_copy(src_hbm.at[tile_blk], sbuf0)
    pltpu.sync_copy(dst_hbm.at[tile_blk], dbuf0)
    gather_desc(0, 0, 0).start()
    for d in efw_descs(0, 0):
        d.start()

    def outer(t, carry):
        for u in range(8):
            g = t * 8 + u
            p = u // 4
            slot = u % 2
            b = u % 4
            if u == 2:
                d1, d2 = idx_descs(tile_blk + t * 2 + 1, 1)
                d1.start()
                d2.start()
            if u == 6:
                @pl.when(t < NT8 - 1)
                def _():
                    d1, d2 = idx_descs(tile_blk + t * 2 + 2, 0)
                    d1.start()
                    d2.start()
            gather_desc(p, b, slot).wait()
            for d in efw_descs(g, slot):
                d.wait()
            if u == 3:
                d1, d2 = idx_descs(tile_blk + t * 2 + 1, 1)
                d1.wait()
                d2.wait()
            if u == 7:
                @pl.when(t < NT8 - 1)
                def _():
                    d1, d2 = idx_descs(tile_blk + t * 2 + 2, 0)
                    d1.wait()
                    d2.wait()
                    gather_desc(0, 0, 0).start()
                    for d in efw_descs(g + 1, 0):
                        d.start()
            else:
                un = u + 1
                gather_desc(un // 4, un % 4, un % 2).start()
                for d in efw_descs(g + 1, un % 2):
                    d.start()
            if u >= 2:
                scat_desc((u - 2) // 4, (u - 2) % 4, slot).wait()
            else:
                @pl.when(t > 0)
                def _():
                    scat_desc(1, (u + 6) % 4, slot).wait()
            compute(slot)
            pltpu.async_copy(outs[slot], acc_sh.at[dbufs[p].at[b]], ssem[slot],
                             add=True)
        return carry

    lax.fori_loop(0, NT8, outer, 0)
    scat_desc(1, 2, 0).wait()
    scat_desc(1, 3, 1).wait()
    plsc.subcore_barrier()

    def copyout(i, carry):
        pltpu.sync_copy(acc_sh.at[pl.ds(s * RPT + i * TSTAGE, TSTAGE)], out0)
        pltpu.sync_copy(out0, out_hbm.at[c, pl.ds(s * RPT + i * TSTAGE, TSTAGE)])
        return carry

    lax.fori_loop(0, RPT // TSTAGE, copyout, 0)


def kernel(x, efeatures, h0, c0, params, edge_index):
    enc = params["encoder_nodes"]
    dec = params["output"]
    wg_t = jnp.concatenate(
        [params["W_i"]["W"].T, params["W_o"]["W"].T, params["W_u"]["W"].T], axis=1)
    bg = jnp.concatenate(
        [params["W_i"]["b"], params["W_o"]["b"], params["W_u"]["b"]])[None, :]

    wx, table = pl.pallas_call(
        _pre_node_body,
        grid=(N // NBLK,),
        in_specs=[
            pl.BlockSpec((NBLK, 128), lambda i: (i, 0)),
            pl.BlockSpec((NBLK, 64), lambda i: (i, 0)),
            pl.BlockSpec((NBLK, 64), lambda i: (i, 0)),
            pl.BlockSpec((128, 64), lambda i: (0, 0)),
            pl.BlockSpec((1, 64), lambda i: (0, 0)),
            pl.BlockSpec((64, 64), lambda i: (0, 0)),
            pl.BlockSpec((1, 64), lambda i: (0, 0)),
            pl.BlockSpec((64, 64), lambda i: (0, 0)),
            pl.BlockSpec((1, 64), lambda i: (0, 0)),
            pl.BlockSpec((1, 64), lambda i: (0, 0)),
            pl.BlockSpec((1, 64), lambda i: (0, 0)),
            pl.BlockSpec((64, 192), lambda i: (0, 0)),
            pl.BlockSpec((1, 192), lambda i: (0, 0)),
            pl.BlockSpec((64, 64), lambda i: (0, 0)),
        ],
        out_specs=[
            pl.BlockSpec((NBLK, 192), lambda i: (i, 0)),
            pl.BlockSpec((NC, NBLK, 96), lambda i: (0, i, 0)),
        ],
        out_shape=[
            jax.ShapeDtypeStruct((N, 192), jnp.float32),
            jax.ShapeDtypeStruct((NC, TAB_ROWS, 96), jnp.float32),
        ],
    )(x, h0, c0,
      enc["input"]["W"].T, enc["input"]["b"][None, :],
      enc["hidden"][0]["W"].T, enc["hidden"][0]["b"][None, :],
      enc["output"]["W"].T, enc["output"]["b"][None, :],
      enc["ln_g"][None, :], enc["ln_b"][None, :],
      wg_t, bg, params["U_f"]["W"].T)

    ef8 = jnp.pad(efeatures, ((0, E_PAD - E), (0, 0))).reshape(E_PAD // 8, 128)
    wf_t = params["W_f"]["W"].T
    eye8 = jnp.eye(8, dtype=jnp.float32)
    w80 = jnp.kron(eye8, wf_t[:, 0:32])
    w81 = jnp.kron(eye8, wf_t[:, 32:64])
    b80 = jnp.tile(params["W_f"]["b"][0:32], 8)[None, :]
    b81 = jnp.tile(params["W_f"]["b"][32:64], 8)[None, :]
    efw = pl.pallas_call(
        _pre_edge_body,
        grid=(E_PAD // EBLK,),
        in_specs=[
            pl.BlockSpec((EBLK // 8, 128), lambda i: (i, 0)),
            pl.BlockSpec((128, 256), lambda i: (0, 0)),
            pl.BlockSpec((1, 256), lambda i: (0, 0)),
            pl.BlockSpec((128, 256), lambda i: (0, 0)),
            pl.BlockSpec((1, 256), lambda i: (0, 0)),
        ],
        out_specs=pl.BlockSpec((NC, 2, EBLK // 8, 128), lambda i: (0, 0, i, 0)),
        out_shape=jax.ShapeDtypeStruct((NC, 2, E_PAD // 8, 128), jnp.float32),
    )(ef8, w80, b80, w81, b81)

    src = jnp.pad(edge_index[0], (0, E_PAD - E)).reshape(E_PAD // (4 * CH), 4 * CH)
    dst = jnp.pad(edge_index[1], (0, E_PAD - E),
                  constant_values=N).reshape(E_PAD // (4 * CH), 4, CH)

    mesh = plsc.VectorSubcoreMesh(core_axis_name="c", subcore_axis_name="s")
    acc = pl.kernel(
        _sc_body,
        out_type=jax.ShapeDtypeStruct((NC, ACC_ROWS, 64), jnp.float32),
        mesh=mesh,
        scratch_types=[
            pltpu.VMEM((4 * CH,), jnp.int32),
            pltpu.VMEM((4 * CH,), jnp.int32),
            pltpu.VMEM((4, CH), jnp.int32),
            pltpu.VMEM((4, CH), jnp.int32),
            pltpu.VMEM((CH, 96), jnp.float32),
            pltpu.VMEM((CH, 96), jnp.float32),
            pltpu.VMEM((2, CH // 8, 128), jnp.float32),
            pltpu.VMEM((2, CH // 8, 128), jnp.float32),
            pltpu.VMEM((CH, 64), jnp.float32),
            pltpu.VMEM((CH, 64), jnp.float32),
            pltpu.VMEM_SHARED((TAB_ROWS, 96), jnp.float32),
            pltpu.VMEM_SHARED((ACC_ROWS, 64), jnp.float32),
            pltpu.SemaphoreType.DMA,
            pltpu.SemaphoreType.DMA,
            pltpu.SemaphoreType.DMA,
            pltpu.SemaphoreType.DMA,
            pltpu.SemaphoreType.DMA,
            pltpu.SemaphoreType.DMA,
            pltpu.SemaphoreType.DMA,
            pltpu.SemaphoreType.DMA,
        ],
        compiler_params=pltpu.CompilerParams(use_tc_tiling_on_sc=False),
    )(table, src, dst, efw)

    y = pl.pallas_call(
        _post_body,
        grid=(N // NBLK,),
        in_specs=[
            pl.BlockSpec((NC, NBLK, 64), lambda i: (0, i, 0)),
            pl.BlockSpec((NBLK, 192), lambda i: (i, 0)),
            pl.BlockSpec((64, 64), lambda i: (0, 0)),
            pl.BlockSpec((64, 64), lambda i: (0, 0)),
            pl.BlockSpec((64, 64), lambda i: (0, 0)),
            pl.BlockSpec((64, 64), lambda i: (0, 0)),
            pl.BlockSpec((1, 64), lambda i: (0, 0)),
            pl.BlockSpec((64, 64), lambda i: (0, 0)),
            pl.BlockSpec((1, 64), lambda i: (0, 0)),
            pl.BlockSpec((64, 2), lambda i: (0, 0)),
            pl.BlockSpec((1, 2), lambda i: (0, 0)),
        ],
        out_specs=pl.BlockSpec((NBLK, 2), lambda i: (i, 0)),
        out_shape=jax.ShapeDtypeStruct((N, 2), jnp.float32),
    )(acc, wx,
      params["U_i"]["W"].T, params["U_o"]["W"].T, params["U_u"]["W"].T,
      dec["input"]["W"].T, dec["input"]["b"][None, :],
      dec["hidden"][0]["W"].T, dec["hidden"][0]["b"][None, :],
      dec["output"]["W"].T, dec["output"]["b"][None, :])
    return y


# speedup vs baseline: 1.2570x; 1.2570x over previous
import functools
import jax
import jax.numpy as jnp
from jax import lax
from jax.experimental import pallas as pl
from jax.experimental.pallas import tpu as pltpu
from jax.experimental.pallas import tpu_sc as plsc

N = 10000
E = 320000
CH = 64
NS = 16
NC = 2
NCHUNK = 320
EPT = NCHUNK * CH
E_PAD = EPT * NS
NB_T = NCHUNK // 4
NT8 = NCHUNK // 8
ACC_ROWS = 10240
RPT = ACC_ROWS // NS
TAB_ROWS = 10240
TRPT = TAB_ROWS // NS
TSTAGE = 64
NBLK = 2000
EBLK = 2048


def _leaky(v):
    return jnp.where(v >= 0.0, v, 0.01 * v)


def _sigmoid(v):
    return 1.0 / (1.0 + jnp.exp(-v))


def _pre_node_body(x_ref, h0_ref, c0_ref, wi_t, bi, wh_t, bh, wo_t, bo,
                   ln_g, ln_b, wg_t, bg, uf_t, wx_ref, tab_ref):
    xb = x_ref[...]
    f = _leaky(jnp.dot(xb, wi_t[...], preferred_element_type=jnp.float32) + bi[...])
    f = _leaky(jnp.dot(f, wh_t[...], preferred_element_type=jnp.float32) + bh[...])
    f = jnp.dot(f, wo_t[...], preferred_element_type=jnp.float32) + bo[...]
    mu = jnp.mean(f, axis=-1, keepdims=True)
    var = jnp.mean((f - mu) * (f - mu), axis=-1, keepdims=True)
    xe = (f - mu) / jnp.sqrt(var + 1e-5) * ln_g[...] + ln_b[...]
    wx_ref[...] = jnp.dot(xe, wg_t[...], preferred_element_type=jnp.float32) + bg[...]
    h0b = h0_ref[...]
    hu = jnp.dot(h0b, uf_t[...], preferred_element_type=jnp.float32)
    sc = _sigmoid(c0_ref[...])
    tab_ref[0] = jnp.concatenate([h0b[:, 0:32], hu[:, 0:32], sc[:, 0:32]], axis=-1)
    tab_ref[1] = jnp.concatenate([h0b[:, 32:64], hu[:, 32:64], sc[:, 32:64]], axis=-1)


def _pre_edge_body(ef_ref, w40, b40, w41, b41, efw_ref):
    e4 = ef_ref[...]
    efw_ref[0] = jnp.dot(e4, w40[...], preferred_element_type=jnp.float32) + b40[...]
    efw_ref[1] = jnp.dot(e4, w41[...], preferred_element_type=jnp.float32) + b41[...]


def _post_body(acc_ref, wx_ref, ui_t, uo_t, uu_t, w1_t, b1, w2_t, b2, w3_t, b3,
               y_ref):
    a0 = acc_ref[0]
    a1 = acc_ref[1]
    s = jnp.concatenate([a0[:, 0:32], a1[:, 0:32]], axis=-1)
    fc = jnp.concatenate([a0[:, 32:64], a1[:, 32:64]], axis=-1)
    wx = wx_ref[...]
    i_g = _sigmoid(wx[:, 0:64] + jnp.dot(s, ui_t[...], preferred_element_type=jnp.float32))
    o_g = _sigmoid(wx[:, 64:128] + jnp.dot(s, uo_t[...], preferred_element_type=jnp.float32))
    u = jnp.tanh(wx[:, 128:192] + jnp.dot(s, uu_t[...], preferred_element_type=jnp.float32))
    c = fc + i_g * u
    h = o_g * jnp.tanh(c)
    f = _leaky(jnp.dot(h, w1_t[...], preferred_element_type=jnp.float32) + b1[...])
    f = _leaky(jnp.dot(f, w2_t[...], preferred_element_type=jnp.float32) + b2[...])
    y_ref[...] = jnp.dot(f, w3_t[...], preferred_element_type=jnp.float32) + b3[...]


def _sc_body(tab_hbm, src_hbm, dst_hbm, efw_hbm, out_hbm,
             sbuf0, sbuf1, dbuf0, dbuf1, rows0, rows1, efw0, efw1, out0, out1,
             tab_sh, acc_sh, is0, is1, gs0, gs1, es0, es1, ss0, ss1):
    c = lax.axis_index("c")
    s = lax.axis_index("s")
    sbufs = [sbuf0, sbuf1]
    dbufs = [dbuf0, dbuf1]
    rows = [rows0, rows1]
    efws = [efw0, efw1]
    outs = [out0, out1]
    isem = [is0, is1]
    gsem = [gs0, gs1]
    esem = [es0, es1]
    ssem = [ss0, ss1]
    tile_blk = s * NB_T
    tile_edge = s * EPT

    def zrow(i, carry):
        for k in range(4):
            out0[i, pl.ds(16 * k, 16)] = jnp.zeros((16,), jnp.float32)
        return carry

    lax.fori_loop(0, CH, zrow, 0)

    def zcopy(i, carry):
        pltpu.sync_copy(out0, acc_sh.at[pl.ds(s * RPT + i * TSTAGE, TSTAGE)])
        return carry

    lax.fori_loop(0, RPT // TSTAGE, zcopy, 0)

    def stage(i, carry):
        pltpu.sync_copy(tab_hbm.at[c, pl.ds(s * TRPT + i * TSTAGE, TSTAGE)], rows0)
        pltpu.sync_copy(rows0, tab_sh.at[pl.ds(s * TRPT + i * TSTAGE, TSTAGE)])
        return carry

    lax.fori_loop(0, TRPT // TSTAGE, stage, 0)
    plsc.subcore_barrier()

    def idx_descs(gblk, p):
        return (pltpu.make_async_copy(src_hbm.at[gblk], sbufs[p], isem[p]),
                pltpu.make_async_copy(dst_hbm.at[gblk], dbufs[p], isem[p]))

    def gather_desc(p, b, slot):
        return pltpu.make_async_copy(
            tab_sh.at[sbufs[p].at[pl.ds(b * CH, CH)]], rows[slot], gsem[slot])

    def efw_descs(g, slot):
        return [pltpu.make_async_copy(
            efw_hbm.at[c, pl.ds(tile_edge // 4 + g * (CH // 4), CH // 4)],
            efws[slot], esem[slot])]

    def scat_desc(p, b, slot):
        return pltpu.make_async_copy(outs[slot], acc_sh.at[dbufs[p].at[b]],
                                     ssem[slot])

    def compute(slot):
        @plsc.parallel_loop(0, CH // 4, unroll=2)
        def erow(i):
            for e in range(4):
                j = i * 4 + e
                for k in range(2):
                    off = 16 * k
                    h0v = rows[slot][j, pl.ds(off, 16)]
                    huv = rows[slot][j, pl.ds(32 + off, 16)]
                    scv = rows[slot][j, pl.ds(64 + off, 16)]
                    ew = efws[slot][i, pl.ds(e * 32 + off, 16)]
                    sg = 1.0 / (1.0 + jnp.exp(-(ew + huv)))
                    outs[slot][j, pl.ds(off, 16)] = h0v
                    outs[slot][j, pl.ds(32 + off, 16)] = sg * scv

    pltpu.sync_copy(src_hbm.at[tile_blk], sbuf0)
    pltpu.sync_copy(dst_hbm.at[tile_blk], dbuf0)
    gather_desc(0, 0, 0).start()
    for d in efw_descs(0, 0):
        d.start()

    def outer(t, carry):
        for u in range(8):
            g = t * 8 + u
            p = u // 4
            slot = u % 2
            b = u % 4
            if u == 2:
                d1, d2 = idx_descs(tile_blk + t * 2 + 1, 1)
                d1.start()
                d2.start()
            if u == 6:
                @pl.when(t < NT8 - 1)
                def _():
                    d1, d2 = idx_descs(tile_blk + t * 2 + 2, 0)
                    d1.start()
                    d2.start()
            gather_desc(p, b, slot).wait()
            for d in efw_descs(g, slot):
                d.wait()
            if u == 3:
                d1, d2 = idx_descs(tile_blk + t * 2 + 1, 1)
                d1.wait()
                d2.wait()
            if u == 7:
                @pl.when(t < NT8 - 1)
                def _():
                    d1, d2 = idx_descs(tile_blk + t * 2 + 2, 0)
                    d1.wait()
                    d2.wait()
                    gather_desc(0, 0, 0).start()
                    for d in efw_descs(g + 1, 0):
                        d.start()
            else:
                un = u + 1
                gather_desc(un // 4, un % 4, un % 2).start()
                for d in efw_descs(g + 1, un % 2):
                    d.start()
            if u >= 2:
                scat_desc((u - 2) // 4, (u - 2) % 4, slot).wait()
            else:
                @pl.when(t > 0)
                def _():
                    scat_desc(1, (u + 6) % 4, slot).wait()
            compute(slot)
            pltpu.async_copy(outs[slot], acc_sh.at[dbufs[p].at[b]], ssem[slot],
                             add=True)
        return carry

    lax.fori_loop(0, NT8, outer, 0)
    scat_desc(1, 2, 0).wait()
    scat_desc(1, 3, 1).wait()
    plsc.subcore_barrier()

    def copyout(i, carry):
        pltpu.sync_copy(acc_sh.at[pl.ds(s * RPT + i * TSTAGE, TSTAGE)], out0)
        pltpu.sync_copy(out0, out_hbm.at[c, pl.ds(s * RPT + i * TSTAGE, TSTAGE)])
        return carry

    lax.fori_loop(0, RPT // TSTAGE, copyout, 0)


def kernel(x, efeatures, h0, c0, params, edge_index):
    enc = params["encoder_nodes"]
    dec = params["output"]
    wg_t = jnp.concatenate(
        [params["W_i"]["W"].T, params["W_o"]["W"].T, params["W_u"]["W"].T], axis=1)
    bg = jnp.concatenate(
        [params["W_i"]["b"], params["W_o"]["b"], params["W_u"]["b"]])[None, :]

    wx, table = pl.pallas_call(
        _pre_node_body,
        grid=(N // NBLK,),
        in_specs=[
            pl.BlockSpec((NBLK, 128), lambda i: (i, 0)),
            pl.BlockSpec((NBLK, 64), lambda i: (i, 0)),
            pl.BlockSpec((NBLK, 64), lambda i: (i, 0)),
            pl.BlockSpec((128, 64), lambda i: (0, 0)),
            pl.BlockSpec((1, 64), lambda i: (0, 0)),
            pl.BlockSpec((64, 64), lambda i: (0, 0)),
            pl.BlockSpec((1, 64), lambda i: (0, 0)),
            pl.BlockSpec((64, 64), lambda i: (0, 0)),
            pl.BlockSpec((1, 64), lambda i: (0, 0)),
            pl.BlockSpec((1, 64), lambda i: (0, 0)),
            pl.BlockSpec((1, 64), lambda i: (0, 0)),
            pl.BlockSpec((64, 192), lambda i: (0, 0)),
            pl.BlockSpec((1, 192), lambda i: (0, 0)),
            pl.BlockSpec((64, 64), lambda i: (0, 0)),
        ],
        out_specs=[
            pl.BlockSpec((NBLK, 192), lambda i: (i, 0)),
            pl.BlockSpec((NC, NBLK, 96), lambda i: (0, i, 0)),
        ],
        out_shape=[
            jax.ShapeDtypeStruct((N, 192), jnp.float32),
            jax.ShapeDtypeStruct((NC, TAB_ROWS, 96), jnp.float32),
        ],
    )(x, h0, c0,
      enc["input"]["W"].T, enc["input"]["b"][None, :],
      enc["hidden"][0]["W"].T, enc["hidden"][0]["b"][None, :],
      enc["output"]["W"].T, enc["output"]["b"][None, :],
      enc["ln_g"][None, :], enc["ln_b"][None, :],
      wg_t, bg, params["U_f"]["W"].T)

    ef4 = jnp.pad(efeatures, ((0, E_PAD - E), (0, 0))).reshape(E_PAD // 4, 64)
    wf_t = params["W_f"]["W"].T
    eye4 = jnp.eye(4, dtype=jnp.float32)
    w40 = jnp.kron(eye4, wf_t[:, 0:32])
    w41 = jnp.kron(eye4, wf_t[:, 32:64])
    b40 = jnp.tile(params["W_f"]["b"][0:32], 4)[None, :]
    b41 = jnp.tile(params["W_f"]["b"][32:64], 4)[None, :]
    efw = pl.pallas_call(
        _pre_edge_body,
        grid=(E_PAD // EBLK,),
        in_specs=[
            pl.BlockSpec((EBLK // 4, 64), lambda i: (i, 0)),
            pl.BlockSpec((64, 128), lambda i: (0, 0)),
            pl.BlockSpec((1, 128), lambda i: (0, 0)),
            pl.BlockSpec((64, 128), lambda i: (0, 0)),
            pl.BlockSpec((1, 128), lambda i: (0, 0)),
        ],
        out_specs=pl.BlockSpec((NC, EBLK // 4, 128), lambda i: (0, i, 0)),
        out_shape=jax.ShapeDtypeStruct((NC, E_PAD // 4, 128), jnp.float32),
    )(ef4, w40, b40, w41, b41)

    src = jnp.pad(edge_index[0], (0, E_PAD - E)).reshape(E_PAD // (4 * CH), 4 * CH)
    dst = jnp.pad(edge_index[1], (0, E_PAD - E),
                  constant_values=N).reshape(E_PAD // (4 * CH), 4, CH)

    mesh = plsc.VectorSubcoreMesh(core_axis_name="c", subcore_axis_name="s")
    acc = pl.kernel(
        _sc_body,
        out_type=jax.ShapeDtypeStruct((NC, ACC_ROWS, 64), jnp.float32),
        mesh=mesh,
        scratch_types=[
            pltpu.VMEM((4 * CH,), jnp.int32),
            pltpu.VMEM((4 * CH,), jnp.int32),
            pltpu.VMEM((4, CH), jnp.int32),
            pltpu.VMEM((4, CH), jnp.int32),
            pltpu.VMEM((CH, 96), jnp.float32),
            pltpu.VMEM((CH, 96), jnp.float32),
            pltpu.VMEM((CH // 4, 128), jnp.float32),
            pltpu.VMEM((CH // 4, 128), jnp.float32),
            pltpu.VMEM((CH, 64), jnp.float32),
            pltpu.VMEM((CH, 64), jnp.float32),
            pltpu.VMEM_SHARED((TAB_ROWS, 96), jnp.float32),
            pltpu.VMEM_SHARED((ACC_ROWS, 64), jnp.float32),
            pltpu.SemaphoreType.DMA,
            pltpu.SemaphoreType.DMA,
            pltpu.SemaphoreType.DMA,
            pltpu.SemaphoreType.DMA,
            pltpu.SemaphoreType.DMA,
            pltpu.SemaphoreType.DMA,
            pltpu.SemaphoreType.DMA,
            pltpu.SemaphoreType.DMA,
        ],
        compiler_params=pltpu.CompilerParams(use_tc_tiling_on_sc=False),
    )(table, src, dst, efw)

    y = pl.pallas_call(
        _post_body,
        grid=(N // NBLK,),
        in_specs=[
            pl.BlockSpec((NC, NBLK, 64), lambda i: (0, i, 0)),
            pl.BlockSpec((NBLK, 192), lambda i: (i, 0)),
            pl.BlockSpec((64, 64), lambda i: (0, 0)),
            pl.BlockSpec((64, 64), lambda i: (0, 0)),
            pl.BlockSpec((64, 64), lambda i: (0, 0)),
            pl.BlockSpec((64, 64), lambda i: (0, 0)),
            pl.BlockSpec((1, 64), lambda i: (0, 0)),
            pl.BlockSpec((64, 64), lambda i: (0, 0)),
            pl.BlockSpec((1, 64), lambda i: (0, 0)),
            pl.BlockSpec((64, 2), lambda i: (0, 0)),
            pl.BlockSpec((1, 2), lambda i: (0, 0)),
        ],
        out_specs=pl.BlockSpec((NBLK, 2), lambda i: (i, 0)),
        out_shape=jax.ShapeDtypeStruct((N, 2), jnp.float32),
    )(acc, wx,
      params["U_i"]["W"].T, params["U_o"]["W"].T, params["U_u"]["W"].T,
      dec["input"]["W"].T, dec["input"]["b"][None, :],
      dec["hidden"][0]["W"].T, dec["hidden"][0]["b"][None, :],
      dec["output"]["W"].T, dec["output"]["b"][None, :])
    return y


# speedup vs baseline: 1.2715x; 1.0115x over previous
import functools
import jax
import jax.numpy as jnp
from jax import lax
from jax.experimental import pallas as pl
from jax.experimental.pallas import tpu as pltpu
from jax.experimental.pallas import tpu_sc as plsc

N = 10000
E = 320000
CH = 64
NS = 16
NC = 2
NCHUNK = 320
EPT = NCHUNK * CH
E_PAD = EPT * NS
NB_T = NCHUNK // 4
NT8 = NCHUNK // 8
ACC_ROWS = 10240
RPT = ACC_ROWS // NS
TAB_ROWS = 10240
TRPT = TAB_ROWS // NS
TSTAGE = 64
NBLK = 2000
EBLK = 2048


def _leaky(v):
    return jnp.where(v >= 0.0, v, 0.01 * v)


def _sigmoid(v):
    return 1.0 / (1.0 + jnp.exp(-v))


def _pre_node_body(x_ref, h0_ref, c0_ref, wi_t, bi, wh_t, bh, wo_t, bo,
                   ln_g, ln_b, wg_t, bg, uf_t, wx_ref, tab_ref):
    xb = x_ref[...]
    f = _leaky(jnp.dot(xb, wi_t[...], preferred_element_type=jnp.float32) + bi[...])
    f = _leaky(jnp.dot(f, wh_t[...], preferred_element_type=jnp.float32) + bh[...])
    f = jnp.dot(f, wo_t[...], preferred_element_type=jnp.float32) + bo[...]
    mu = jnp.mean(f, axis=-1, keepdims=True)
    var = jnp.mean((f - mu) * (f - mu), axis=-1, keepdims=True)
    xe = (f - mu) / jnp.sqrt(var + 1e-5) * ln_g[...] + ln_b[...]
    wx_ref[...] = jnp.dot(xe, wg_t[...], preferred_element_type=jnp.float32) + bg[...]
    h0b = h0_ref[...]
    hu = jnp.dot(h0b, uf_t[...], preferred_element_type=jnp.float32)
    sc = _sigmoid(c0_ref[...])
    tab_ref[0] = jnp.concatenate([h0b[:, 0:32], hu[:, 0:32], sc[:, 0:32]], axis=-1)
    tab_ref[1] = jnp.concatenate([h0b[:, 32:64], hu[:, 32:64], sc[:, 32:64]], axis=-1)


def _pre_edge_body(ef_ref, w40, b40, w41, b41, efw_ref):
    e4 = ef_ref[...]
    efw_ref[0] = jnp.dot(e4, w40[...], preferred_element_type=jnp.float32) + b40[...]
    efw_ref[1] = jnp.dot(e4, w41[...], preferred_element_type=jnp.float32) + b41[...]


def _post_body(acc_ref, wx_ref, ui_t, uo_t, uu_t, w1_t, b1, w2_t, b2, w3_t, b3,
               y_ref):
    a0 = acc_ref[0]
    a1 = acc_ref[1]
    s = jnp.concatenate([a0[:, 0:32], a1[:, 0:32]], axis=-1)
    fc = jnp.concatenate([a0[:, 32:64], a1[:, 32:64]], axis=-1)
    wx = wx_ref[...]
    i_g = _sigmoid(wx[:, 0:64] + jnp.dot(s, ui_t[...], preferred_element_type=jnp.float32))
    o_g = _sigmoid(wx[:, 64:128] + jnp.dot(s, uo_t[...], preferred_element_type=jnp.float32))
    u = jnp.tanh(wx[:, 128:192] + jnp.dot(s, uu_t[...], preferred_element_type=jnp.float32))
    c = fc + i_g * u
    h = o_g * jnp.tanh(c)
    f = _leaky(jnp.dot(h, w1_t[...], preferred_element_type=jnp.float32) + b1[...])
    f = _leaky(jnp.dot(f, w2_t[...], preferred_element_type=jnp.float32) + b2[...])
    y_ref[...] = jnp.dot(f, w3_t[...], preferred_element_type=jnp.float32) + b3[...]


def _sc_body(tab_hbm, src_hbm, dst_hbm, efw_hbm, out_hbm,
             sbuf0, sbuf1, dbuf0, dbuf1, rows0, rows1, efw0, efw1, out0, out1,
             tab_sh, acc_sh, is0, is1, gs0, gs1, es0, es1, ss0, ss1):
    c = lax.axis_index("c")
    s = lax.axis_index("s")
    sbufs = [sbuf0, sbuf1]
    dbufs = [dbuf0, dbuf1]
    rows = [rows0, rows1]
    efws = [efw0, efw1]
    outs = [out0, out1]
    isem = [is0, is1]
    gsem = [gs0, gs1]
    esem = [es0, es1]
    ssem = [ss0, ss1]
    tile_blk = s * NB_T
    tile_edge = s * EPT

    def zrow(i, carry):
        for k in range(4):
            out0[i, pl.ds(16 * k, 16)] = jnp.zeros((16,), jnp.float32)
        return carry

    lax.fori_loop(0, CH, zrow, 0)

    def zcopy(i, carry):
        pltpu.sync_copy(out0, acc_sh.at[pl.ds(s * RPT + i * TSTAGE, TSTAGE)])
        return carry

    lax.fori_loop(0, RPT // TSTAGE, zcopy, 0)

    def stage(i, carry):
        pltpu.sync_copy(tab_hbm.at[c, pl.ds(s * TRPT + i * TSTAGE, TSTAGE)], rows0)
        pltpu.sync_copy(rows0, tab_sh.at[pl.ds(s * TRPT + i * TSTAGE, TSTAGE)])
        return carry

    lax.fori_loop(0, TRPT // TSTAGE, stage, 0)
    plsc.subcore_barrier()

    def idx_descs(gblk, p):
        return (pltpu.make_async_copy(src_hbm.at[gblk], sbufs[p], isem[p]),
                pltpu.make_async_copy(dst_hbm.at[gblk], dbufs[p], isem[p]))

    def gather_desc(p, b, slot):
        return pltpu.make_async_copy(
            tab_sh.at[sbufs[p].at[pl.ds(b * CH, CH)]], rows[slot], gsem[slot])

    def efw_descs(g, slot):
        return [pltpu.make_async_copy(
            efw_hbm.at[c, pl.ds(tile_edge // 4 + g * (CH // 4), CH // 4)],
            efws[slot], esem[slot])]

    def scat_desc(p, b, slot):
        return pltpu.make_async_copy(outs[slot], acc_sh.at[dbufs[p].at[b]],
                                     ssem[slot])

    def compute(slot):
        @plsc.parallel_loop(0, CH // 4, unroll=4)
        def erow(i):
            for e in range(4):
                j = i * 4 + e
                for k in range(2):
                    off = 16 * k
                    h0v = rows[slot][j, pl.ds(off, 16)]
                    huv = rows[slot][j, pl.ds(32 + off, 16)]
                    scv = rows[slot][j, pl.ds(64 + off, 16)]
                    ew = efws[slot][i, pl.ds(e * 32 + off, 16)]
                    sg = 1.0 / (1.0 + jnp.exp(-(ew + huv)))
                    outs[slot][j, pl.ds(off, 16)] = h0v
                    outs[slot][j, pl.ds(32 + off, 16)] = sg * scv

    pltpu.sync_copy(src_hbm.at[tile_blk], sbuf0)
    pltpu.sync_copy(dst_hbm.at[tile_blk], dbuf0)
    gather_desc(0, 0, 0).start()
    for d in efw_descs(0, 0):
        d.start()

    def outer(t, carry):
        for u in range(8):
            g = t * 8 + u
            p = u // 4
            slot = u % 2
            b = u % 4
            if u == 2:
                d1, d2 = idx_descs(tile_blk + t * 2 + 1, 1)
                d1.start()
                d2.start()
            if u == 6:
                @pl.when(t < NT8 - 1)
                def _():
                    d1, d2 = idx_descs(tile_blk + t * 2 + 2, 0)
                    d1.start()
                    d2.start()
            gather_desc(p, b, slot).wait()
            for d in efw_descs(g, slot):
                d.wait()
            if u == 3:
                d1, d2 = idx_descs(tile_blk + t * 2 + 1, 1)
                d1.wait()
                d2.wait()
            if u == 7:
                @pl.when(t < NT8 - 1)
                def _():
                    d1, d2 = idx_descs(tile_blk + t * 2 + 2, 0)
                    d1.wait()
                    d2.wait()
                    gather_desc(0, 0, 0).start()
                    for d in efw_descs(g + 1, 0):
                        d.start()
            else:
                un = u + 1
                gather_desc(un // 4, un % 4, un % 2).start()
                for d in efw_descs(g + 1, un % 2):
                    d.start()
            if u >= 2:
                scat_desc((u - 2) // 4, (u - 2) % 4, slot).wait()
            else:
                @pl.when(t > 0)
                def _():
                    scat_desc(1, (u + 6) % 4, slot).wait()
            compute(slot)
            pltpu.async_copy(outs[slot], acc_sh.at[dbufs[p].at[b]], ssem[slot],
                             add=True)
        return carry

    lax.fori_loop(0, NT8, outer, 0)
    scat_desc(1, 2, 0).wait()
    scat_desc(1, 3, 1).wait()
    plsc.subcore_barrier()

    def copyout(i, carry):
        pltpu.sync_copy(acc_sh.at[pl.ds(s * RPT + i * TSTAGE, TSTAGE)], out0)
        pltpu.sync_copy(out0, out_hbm.at[c, pl.ds(s * RPT + i * TSTAGE, TSTAGE)])
        return carry

    lax.fori_loop(0, RPT // TSTAGE, copyout, 0)


def kernel(x, efeatures, h0, c0, params, edge_index):
    enc = params["encoder_nodes"]
    dec = params["output"]
    wg_t = jnp.concatenate(
        [params["W_i"]["W"].T, params["W_o"]["W"].T, params["W_u"]["W"].T], axis=1)
    bg = jnp.concatenate(
        [params["W_i"]["b"], params["W_o"]["b"], params["W_u"]["b"]])[None, :]

    wx, table = pl.pallas_call(
        _pre_node_body,
        grid=(N // NBLK,),
        in_specs=[
            pl.BlockSpec((NBLK, 128), lambda i: (i, 0)),
            pl.BlockSpec((NBLK, 64), lambda i: (i, 0)),
            pl.BlockSpec((NBLK, 64), lambda i: (i, 0)),
            pl.BlockSpec((128, 64), lambda i: (0, 0)),
            pl.BlockSpec((1, 64), lambda i: (0, 0)),
            pl.BlockSpec((64, 64), lambda i: (0, 0)),
            pl.BlockSpec((1, 64), lambda i: (0, 0)),
            pl.BlockSpec((64, 64), lambda i: (0, 0)),
            pl.BlockSpec((1, 64), lambda i: (0, 0)),
            pl.BlockSpec((1, 64), lambda i: (0, 0)),
            pl.BlockSpec((1, 64), lambda i: (0, 0)),
            pl.BlockSpec((64, 192), lambda i: (0, 0)),
            pl.BlockSpec((1, 192), lambda i: (0, 0)),
            pl.BlockSpec((64, 64), lambda i: (0, 0)),
        ],
        out_specs=[
            pl.BlockSpec((NBLK, 192), lambda i: (i, 0)),
            pl.BlockSpec((NC, NBLK, 96), lambda i: (0, i, 0)),
        ],
        out_shape=[
            jax.ShapeDtypeStruct((N, 192), jnp.float32),
            jax.ShapeDtypeStruct((NC, TAB_ROWS, 96), jnp.float32),
        ],
    )(x, h0, c0,
      enc["input"]["W"].T, enc["input"]["b"][None, :],
      enc["hidden"][0]["W"].T, enc["hidden"][0]["b"][None, :],
      enc["output"]["W"].T, enc["output"]["b"][None, :],
      enc["ln_g"][None, :], enc["ln_b"][None, :],
      wg_t, bg, params["U_f"]["W"].T)

    ef4 = jnp.pad(efeatures, ((0, E_PAD - E), (0, 0))).reshape(E_PAD // 4, 64)
    wf_t = params["W_f"]["W"].T
    eye4 = jnp.eye(4, dtype=jnp.float32)
    w40 = jnp.kron(eye4, wf_t[:, 0:32])
    w41 = jnp.kron(eye4, wf_t[:, 32:64])
    b40 = jnp.tile(params["W_f"]["b"][0:32], 4)[None, :]
    b41 = jnp.tile(params["W_f"]["b"][32:64], 4)[None, :]
    efw = pl.pallas_call(
        _pre_edge_body,
        grid=(E_PAD // EBLK,),
        in_specs=[
            pl.BlockSpec((EBLK // 4, 64), lambda i: (i, 0)),
            pl.BlockSpec((64, 128), lambda i: (0, 0)),
            pl.BlockSpec((1, 128), lambda i: (0, 0)),
            pl.BlockSpec((64, 128), lambda i: (0, 0)),
            pl.BlockSpec((1, 128), lambda i: (0, 0)),
        ],
        out_specs=pl.BlockSpec((NC, EBLK // 4, 128), lambda i: (0, i, 0)),
        out_shape=jax.ShapeDtypeStruct((NC, E_PAD // 4, 128), jnp.float32),
    )(ef4, w40, b40, w41, b41)

    src = jnp.pad(edge_index[0], (0, E_PAD - E)).reshape(E_PAD // (4 * CH), 4 * CH)
    dst = jnp.pad(edge_index[1], (0, E_PAD - E),
                  constant_values=N).reshape(E_PAD // (4 * CH), 4, CH)

    mesh = plsc.VectorSubcoreMesh(core_axis_name="c", subcore_axis_name="s")
    acc = pl.kernel(
        _sc_body,
        out_type=jax.ShapeDtypeStruct((NC, ACC_ROWS, 64), jnp.float32),
        mesh=mesh,
        scratch_types=[
            pltpu.VMEM((4 * CH,), jnp.int32),
            pltpu.VMEM((4 * CH,), jnp.int32),
            pltpu.VMEM((4, CH), jnp.int32),
            pltpu.VMEM((4, CH), jnp.int32),
            pltpu.VMEM((CH, 96), jnp.float32),
            pltpu.VMEM((CH, 96), jnp.float32),
            pltpu.VMEM((CH // 4, 128), jnp.float32),
            pltpu.VMEM((CH // 4, 128), jnp.float32),
            pltpu.VMEM((CH, 64), jnp.float32),
            pltpu.VMEM((CH, 64), jnp.float32),
            pltpu.VMEM_SHARED((TAB_ROWS, 96), jnp.float32),
            pltpu.VMEM_SHARED((ACC_ROWS, 64), jnp.float32),
            pltpu.SemaphoreType.DMA,
            pltpu.SemaphoreType.DMA,
            pltpu.SemaphoreType.DMA,
            pltpu.SemaphoreType.DMA,
            pltpu.SemaphoreType.DMA,
            pltpu.SemaphoreType.DMA,
            pltpu.SemaphoreType.DMA,
            pltpu.SemaphoreType.DMA,
        ],
        compiler_params=pltpu.CompilerParams(use_tc_tiling_on_sc=False),
    )(table, src, dst, efw)

    y = pl.pallas_call(
        _post_body,
        grid=(N // NBLK,),
        in_specs=[
            pl.BlockSpec((NC, NBLK, 64), lambda i: (0, i, 0)),
            pl.BlockSpec((NBLK, 192), lambda i: (i, 0)),
            pl.BlockSpec((64, 64), lambda i: (0, 0)),
            pl.BlockSpec((64, 64), lambda i: (0, 0)),
            pl.BlockSpec((64, 64), lambda i: (0, 0)),
            pl.BlockSpec((64, 64), lambda i: (0, 0)),
            pl.BlockSpec((1, 64), lambda i: (0, 0)),
            pl.BlockSpec((64, 64), lambda i: (0, 0)),
            pl.BlockSpec((1, 64), lambda i: (0, 0)),
            pl.BlockSpec((64, 2), lambda i: (0, 0)),
            pl.BlockSpec((1, 2), lambda i: (0, 0)),
        ],
        out_specs=pl.BlockSpec((NBLK, 2), lambda i: (i, 0)),
        out_shape=jax.ShapeDtypeStruct((N, 2), jnp.float32),
    )(acc, wx,
      params["U_i"]["W"].T, params["U_o"]["W"].T, params["U_u"]["W"].T,
      dec["input"]["W"].T, dec["input"]["b"][None, :],
      dec["hidden"][0]["W"].T, dec["hidden"][0]["b"][None, :],
      dec["output"]["W"].T, dec["output"]["b"][None, :])
    return y
